# Initial kernel scaffold; baseline (speedup 1.0000x reference)
import jax
import jax.numpy as jnp
from jax.experimental import pallas as pl


def kernel(scores, targets, lengths):
    raise NotImplementedError("write your pallas kernel here")


# speedup vs baseline: 2.4087x; 2.4087x over previous
import functools

import jax
import jax.numpy as jnp
from jax import lax
from jax.experimental import pallas as pl
from jax.experimental.pallas import tpu as pltpu
from jax.experimental.pallas import tpu_sc as plsc

_B = 32
_T = 512
_K = 64
_START = 62
_END = 63

_NC = 2
_NS = 16
_NW = _NC * _NS
_L = 16
_GATHER_CHUNK = 128


def _gold_body(scores_hbm, targets_hbm, lengths_hbm, out_hbm,
               tgt_v, idx_v, val_v, len_v, acc_v, sem):
    w = lax.axis_index("s") * _NC + lax.axis_index("c")
    pltpu.sync_copy(targets_hbm.at[w], tgt_v)
    pltpu.sync_copy(lengths_hbm.at[w], len_v)
    lb = len_v[...]

    base = w * (_T * _K * _K)
    for c in range(_T // _L):
        tvec = lax.iota(jnp.int32, _L) + (c * _L)
        tgt = tgt_v[pl.ds(c * _L, _L)]
        idx_v[pl.ds(c * _L, _L)] = base + tvec * (_K * _K) + tgt

    descs = []
    for j in range(_T // _GATHER_CHUNK):
        sl = pl.ds(j * _GATHER_CHUNK, _GATHER_CHUNK)
        descs.append(pltpu.async_copy(scores_hbm.at[idx_v.at[sl]],
                                      val_v.at[sl], sem))
    for d in descs:
        d.wait()

    acc = jnp.zeros((_L,), jnp.float32)
    for c in range(_T // _L):
        tvec = lax.iota(jnp.int32, _L) + (c * _L)
        v = val_v[pl.ds(c * _L, _L)]
        acc = acc + jnp.where(tvec < lb, v, 0.0)
    acc_v[...] = acc
    pltpu.sync_copy(acc_v, out_hbm.at[w])


@functools.cache
def _gold_partials():
    return pl.kernel(
        _gold_body,
        out_type=jax.ShapeDtypeStruct((_NW, _L), jnp.float32),
        mesh=plsc.VectorSubcoreMesh(core_axis_name="c", subcore_axis_name="s",
                                    num_cores=_NC, num_subcores=_NS),
        scratch_types=[
            pltpu.VMEM((_T,), jnp.int32),
            pltpu.VMEM((_T,), jnp.int32),
            pltpu.VMEM((_T,), jnp.float32),
            pltpu.VMEM((_L,), jnp.int32),
            pltpu.VMEM((_L,), jnp.float32),
            pltpu.SemaphoreType.DMA,
        ],
    )


def _fwd_body(len_ref, scores_ref, out_ref, alpha_ref):
    t = pl.program_id(0)
    s = scores_ref[:, 0]

    @pl.when(t == 0)
    def _init():
        alpha_ref[...] = s[:, _START, :]

    @pl.when(t > 0)
    def _step():
        a = alpha_ref[...]
        x = s + a[:, :, None]
        m = jnp.max(x, axis=1)
        e = jnp.exp(x - m[:, None, :])
        new = m + jnp.log(jnp.sum(e, axis=1))
        active = len_ref[...] > t
        alpha_ref[...] = jnp.where(active, new, a)

    @pl.when(t == _T - 1)
    def _fin():
        out_ref[0, 0] = jnp.sum(alpha_ref[:, _END])


def _forward(scores, lengths2d):
    return pl.pallas_call(
        _fwd_body,
        grid=(_T,),
        in_specs=[
            pl.BlockSpec((_B, _K), lambda t: (0, 0)),
            pl.BlockSpec((_B, 1, _K, _K), lambda t: (0, t, 0, 0)),
        ],
        out_specs=pl.BlockSpec(memory_space=pltpu.SMEM),
        out_shape=jax.ShapeDtypeStruct((1, 1), jnp.float32),
        scratch_shapes=[pltpu.VMEM((_B, _K), jnp.float32)],
    )(lengths2d, scores)


@jax.jit
def kernel(scores, targets, lengths):
    flat = scores.reshape(-1)
    len_splat = jnp.broadcast_to(lengths.astype(jnp.int32)[:, None], (_B, _L))
    gold = jnp.sum(_gold_partials()(flat, targets.astype(jnp.int32), len_splat))
    lengths2d = jnp.broadcast_to(lengths.astype(jnp.int32)[:, None], (_B, _K))
    all_paths = _forward(scores, lengths2d)[0, 0]
    return (all_paths - gold) / _B


# speedup vs baseline: 2.6304x; 1.0921x over previous
import functools

import jax
import jax.numpy as jnp
from jax import lax
from jax.experimental import pallas as pl
from jax.experimental.pallas import tpu as pltpu
from jax.experimental.pallas import tpu_sc as plsc

_B = 32
_T = 512
_K = 64
_START = 62
_END = 63

_NC = 2
_NS = 16
_NW = _NC * _NS
_L = 16
_GATHER_CHUNK = 128


def _gold_body(scores_hbm, targets_hbm, lengths_hbm, out_hbm,
               tgt_v, idx_v, val_v, len_v, acc_v, sem):
    w = lax.axis_index("s") * _NC + lax.axis_index("c")
    pltpu.sync_copy(targets_hbm.at[w], tgt_v)
    pltpu.sync_copy(lengths_hbm.at[w], len_v)
    lb = len_v[...]

    base = w * (_T * _K * _K)
    for c in range(_T // _L):
        tvec = lax.iota(jnp.int32, _L) + (c * _L)
        tgt = tgt_v[pl.ds(c * _L, _L)]
        idx_v[pl.ds(c * _L, _L)] = base + tvec * (_K * _K) + tgt

    descs = []
    for j in range(_T // _GATHER_CHUNK):
        sl = pl.ds(j * _GATHER_CHUNK, _GATHER_CHUNK)
        descs.append(pltpu.async_copy(scores_hbm.at[idx_v.at[sl]],
                                      val_v.at[sl], sem))
    for d in descs:
        d.wait()

    acc = jnp.zeros((_L,), jnp.float32)
    for c in range(_T // _L):
        tvec = lax.iota(jnp.int32, _L) + (c * _L)
        v = val_v[pl.ds(c * _L, _L)]
        acc = acc + jnp.where(tvec < lb, v, 0.0)
    acc_v[...] = acc
    pltpu.sync_copy(acc_v, out_hbm.at[w])


@functools.cache
def _gold_partials():
    return pl.kernel(
        _gold_body,
        out_type=jax.ShapeDtypeStruct((_NW, _L), jnp.float32),
        mesh=plsc.VectorSubcoreMesh(core_axis_name="c", subcore_axis_name="s",
                                    num_cores=_NC, num_subcores=_NS),
        scratch_types=[
            pltpu.VMEM((_T,), jnp.int32),
            pltpu.VMEM((_T,), jnp.int32),
            pltpu.VMEM((_T,), jnp.float32),
            pltpu.VMEM((_L,), jnp.int32),
            pltpu.VMEM((_L,), jnp.float32),
            pltpu.SemaphoreType.DMA,
        ],
    )


_C = 8
_TT = 8
_NCHUNK = _B // _C


def _fwd_body(len_smem, len2d_ref, scores_hbm, out_ref,
              buf_ref, alpha_ref, off_ref, sem):
    c = pl.program_id(0)
    lmax = len_smem[c * _C]
    ngrp = (lmax + _TT - 1) // _TT

    def dma(g, slot):
        return pltpu.make_async_copy(
            scores_hbm.at[pl.ds(c * _C, _C), pl.ds(g * _TT, _TT)],
            buf_ref.at[slot], sem.at[slot])

    dma(0, 0).start()

    def group(g, carry):
        slot = lax.rem(g, 2)

        @pl.when(g + 1 < ngrp)
        def _prefetch():
            dma(g + 1, lax.rem(g + 1, 2)).start()

        dma(g, slot).wait()
        block = buf_ref[slot]

        @pl.when(g == 0)
        def _init():
            alpha_ref[...] = block[:, 0, _START, :]
            off_ref[...] = jnp.zeros((_C, _K), jnp.float32)

        for tt in range(_TT):
            t = g * _TT + tt
            x = block[:, tt] + alpha_ref[...][:, :, None]
            p = jnp.sum(jnp.exp(x), axis=1)
            newv = jnp.log(p)
            nm = jnp.max(newv, axis=1, keepdims=True)
            act = (len2d_ref[...] > t) & (t > 0)
            alpha_ref[...] = jnp.where(act, newv - nm, alpha_ref[...])
            off_ref[...] = jnp.where(act, off_ref[...] + nm, off_ref[...])
        return carry

    lax.fori_loop(0, ngrp, group, 0)
    partial = jnp.sum(alpha_ref[:, _END] + off_ref[:, _END])

    @pl.when(c == 0)
    def _first():
        out_ref[0, 0] = partial

    @pl.when(c > 0)
    def _rest():
        out_ref[0, 0] = out_ref[0, 0] + partial


def _forward(scores, lengths, lengths2d):
    return pl.pallas_call(
        _fwd_body,
        grid=(_NCHUNK,),
        in_specs=[
            pl.BlockSpec(memory_space=pltpu.SMEM),
            pl.BlockSpec((_C, _K), lambda c: (c, 0)),
            pl.BlockSpec(memory_space=pl.ANY),
        ],
        out_specs=pl.BlockSpec(memory_space=pltpu.SMEM),
        out_shape=jax.ShapeDtypeStruct((1, 1), jnp.float32),
        scratch_shapes=[
            pltpu.VMEM((2, _C, _TT, _K, _K), jnp.float32),
            pltpu.VMEM((_C, _K), jnp.float32),
            pltpu.VMEM((_C, _K), jnp.float32),
            pltpu.SemaphoreType.DMA((2,)),
        ],
    )(lengths, lengths2d, scores)


@jax.jit
def kernel(scores, targets, lengths):
    flat = scores.reshape(-1)
    len_splat = jnp.broadcast_to(lengths.astype(jnp.int32)[:, None], (_B, _L))
    gold = jnp.sum(_gold_partials()(flat, targets.astype(jnp.int32), len_splat))
    lengths2d = jnp.broadcast_to(lengths.astype(jnp.int32)[:, None], (_B, _K))
    all_paths = _forward(scores, lengths.astype(jnp.int32), lengths2d)[0, 0]
    return (all_paths - gold) / _B
